# baseline (speedup 1.0000x reference)
import jax
import jax.numpy as jnp
from jax.experimental import pallas as pl
from jax.experimental.pallas import tpu as pltpu

_BM = 400


_BS = _BM // 2


def _gcn_kernel(
    x_ref, w1_ref, b1_ref, w2_ref, b2_ref, adj0_ref, adj1_ref, out_ref,
    h_ref, agg_ref
):
    p = pl.program_id(0)
    i = pl.program_id(1)

    @pl.when((p == 0) & (i == 0))
    def _():
        h_ref[...] = (
            jnp.dot(x_ref[...], w1_ref[...], preferred_element_type=jnp.float32)
            + b1_ref[...]
        )

    @pl.when((p == 1) & (i == 0))
    def _():
        h_ref[...] = (
            jnp.dot(agg_ref[...], w2_ref[...], preferred_element_type=jnp.float32)
            + b2_ref[...]
        )

    acc0 = jnp.dot(adj0_ref[...], h_ref[...], preferred_element_type=jnp.float32)
    acc1 = jnp.dot(adj1_ref[...], h_ref[...], preferred_element_type=jnp.float32)

    @pl.when(p == 0)
    def _():
        agg_ref[pl.ds(i * _BM, _BS), :] = jnp.maximum(acc0, 0.0)
        agg_ref[pl.ds(i * _BM + _BS, _BS), :] = jnp.maximum(acc1, 0.0)

    @pl.when(p == 1)
    def _():
        out_ref[pl.ds(0, _BS), :] = acc0
        out_ref[pl.ds(_BS, _BS), :] = acc1


def kernel(x, Adj, W1, b1, W2, b2):
    n, d_in = x.shape
    d_h = W1.shape[1]
    d_out = W2.shape[1]
    return pl.pallas_call(
        _gcn_kernel,
        grid=(2, n // _BM),
        in_specs=[
            pl.BlockSpec((n, d_in), lambda p, i: (0, 0)),
            pl.BlockSpec((d_in, d_h), lambda p, i: (0, 0)),
            pl.BlockSpec((1, d_h), lambda p, i: (0, 0)),
            pl.BlockSpec((d_h, d_out), lambda p, i: (0, 0)),
            pl.BlockSpec((1, d_out), lambda p, i: (0, 0)),
            pl.BlockSpec((_BS, n), lambda p, i: (2 * i, 0)),
            pl.BlockSpec((_BS, n), lambda p, i: (2 * i + 1, 0)),
        ],
        out_specs=pl.BlockSpec((_BM, d_out), lambda p, i: (p * i, 0)),
        out_shape=jax.ShapeDtypeStruct((n, d_out), jnp.float32),
        scratch_shapes=[
            pltpu.VMEM((n, d_h), jnp.float32),
            pltpu.VMEM((n, d_h), jnp.float32),
        ],
        compiler_params=pltpu.CompilerParams(
            dimension_semantics=("arbitrary", "arbitrary"),
        ),
    )(x, W1, b1.reshape(1, -1), W2, b2.reshape(1, -1), Adj, Adj)


# speedup vs baseline: 1.0071x; 1.0071x over previous
import jax
import jax.numpy as jnp
from jax.experimental import pallas as pl
from jax.experimental.pallas import tpu as pltpu

_BM = 400


def _gcn_kernel(
    x_ref, w1_ref, b1_ref, w2_ref, b2_ref, adj_ref, out_ref, h_ref, agg_ref
):
    p = pl.program_id(0)
    i = pl.program_id(1)

    @pl.when((p == 0) & (i == 0))
    def _():
        h_ref[...] = (
            jnp.dot(x_ref[...], w1_ref[...], preferred_element_type=jnp.float32)
            + b1_ref[...]
        )

    @pl.when((p == 1) & (i == 0))
    def _():
        h_ref[...] = (
            jnp.dot(agg_ref[...], w2_ref[...], preferred_element_type=jnp.float32)
            + b2_ref[...]
        )

    acc = jnp.dot(adj_ref[...], h_ref[...], preferred_element_type=jnp.float32)

    @pl.when(p == 0)
    def _():
        agg_ref[pl.ds(i * _BM, _BM), :] = jnp.maximum(acc, 0.0)

    @pl.when(p == 1)
    def _():
        out_ref[...] = acc


def kernel(x, Adj, W1, b1, W2, b2):
    n, d_in = x.shape
    d_h = W1.shape[1]
    d_out = W2.shape[1]
    return pl.pallas_call(
        _gcn_kernel,
        grid=(2, n // _BM),
        in_specs=[
            pl.BlockSpec((n, d_in), lambda p, i: (0, 0)),
            pl.BlockSpec((d_in, d_h), lambda p, i: (0, 0)),
            pl.BlockSpec((1, d_h), lambda p, i: (0, 0)),
            pl.BlockSpec((d_h, d_out), lambda p, i: (0, 0)),
            pl.BlockSpec((1, d_out), lambda p, i: (0, 0)),
            pl.BlockSpec((_BM, n), lambda p, i: (i, 0)),
        ],
        out_specs=pl.BlockSpec((_BM, d_out), lambda p, i: (p * i, 0)),
        out_shape=jax.ShapeDtypeStruct((n, d_out), jnp.float32),
        scratch_shapes=[
            pltpu.VMEM((n, d_h), jnp.float32),
            pltpu.VMEM((n, d_h), jnp.float32),
        ],
        compiler_params=pltpu.CompilerParams(
            dimension_semantics=("arbitrary", "arbitrary"),
        ),
    )(x, W1, b1.reshape(1, -1), W2, b2.reshape(1, -1), Adj)


# speedup vs baseline: 1.0102x; 1.0031x over previous
import jax
import jax.numpy as jnp
from jax.experimental import pallas as pl
from jax.experimental.pallas import tpu as pltpu

_BM = 400


def _gcn_kernel(
    x_ref, w1_ref, b1_ref, w2_ref, b2_ref, adj_ref, out_ref, h_ref, agg_ref
):
    p = pl.program_id(0)
    i = pl.program_id(1)

    @pl.when((p == 0) & (i == 0))
    def _():
        h_ref[...] = (
            jnp.dot(x_ref[...], w1_ref[...], preferred_element_type=jnp.float32)
            + b1_ref[...]
        )

    @pl.when(p == 0)
    def _():
        acc = jnp.dot(adj_ref[...], h_ref[...], preferred_element_type=jnp.float32)
        agg_ref[pl.ds(i * _BM, _BM), :] = (
            jnp.dot(
                jnp.maximum(acc, 0.0),
                w2_ref[...],
                preferred_element_type=jnp.float32,
            )
            + b2_ref[...]
        )

    @pl.when(p == 1)
    def _():
        out_ref[...] = jnp.dot(
            adj_ref[...], agg_ref[...], preferred_element_type=jnp.float32
        )


def kernel(x, Adj, W1, b1, W2, b2):
    n, d_in = x.shape
    d_h = W1.shape[1]
    d_out = W2.shape[1]
    return pl.pallas_call(
        _gcn_kernel,
        grid=(2, n // _BM),
        in_specs=[
            pl.BlockSpec((n, d_in), lambda p, i: (0, 0)),
            pl.BlockSpec((d_in, d_h), lambda p, i: (0, 0)),
            pl.BlockSpec((1, d_h), lambda p, i: (0, 0)),
            pl.BlockSpec((d_h, d_out), lambda p, i: (0, 0)),
            pl.BlockSpec((1, d_out), lambda p, i: (0, 0)),
            pl.BlockSpec((_BM, n), lambda p, i: (i, 0)),
        ],
        out_specs=pl.BlockSpec((_BM, d_out), lambda p, i: (p * i, 0)),
        out_shape=jax.ShapeDtypeStruct((n, d_out), jnp.float32),
        scratch_shapes=[
            pltpu.VMEM((n, d_h), jnp.float32),
            pltpu.VMEM((n, d_h), jnp.float32),
        ],
        compiler_params=pltpu.CompilerParams(
            dimension_semantics=("arbitrary", "arbitrary"),
        ),
    )(x, W1, b1.reshape(1, -1), W2, b2.reshape(1, -1), Adj)
